# baseline (speedup 1.0000x reference)
import functools

import jax
import jax.numpy as jnp
from jax import lax
from jax.experimental import pallas as pl
from jax.experimental.pallas import tpu as pltpu
from jax.experimental.pallas import tpu_sc as plsc

F = 26
V = 100000
E = 32
B = 4096
GEN = F * E
DOM = 4 * E
GH = 256
TN = 2
NW = 32
RPW = GEN // NW


def _sc_gather_t(idxT, t2):
    mesh = plsc.VectorSubcoreMesh(core_axis_name="c", subcore_axis_name="s")
    nc = mesh.num_cores

    @functools.partial(
        pl.kernel,
        out_type=jax.ShapeDtypeStruct((GEN, B), jnp.float32),
        mesh=mesh,
        scratch_types=[
            pltpu.VMEM((V,), jnp.float32),
            pltpu.VMEM((B,), jnp.int32),
            pltpu.VMEM((B,), jnp.float32),
        ],
        compiler_params=pltpu.CompilerParams(needs_layout_passes=False),
    )
    def k(idx_hbm, t2_hbm, out_hbm, row_v, idx_v, ob_v):
        wid = lax.axis_index("s") * nc + lax.axis_index("c")
        for m in range(RPW):
            r = wid * RPW + m
            f = r // E
            if m == 0:
                pltpu.sync_copy(idx_hbm.at[f], idx_v)
            else:
                f_prev = (wid * RPW + m - 1) // E

                @pl.when(f != f_prev)
                def _load_idx(f=f):
                    pltpu.sync_copy(idx_hbm.at[f], idx_v)

            pltpu.sync_copy(t2_hbm.at[r], row_v)

            def body(i, _):
                idx16 = idx_v[pl.ds(i * 16, 16)]
                ob_v[pl.ds(i * 16, 16)] = plsc.load_gather(row_v, [idx16])
                return 0

            lax.fori_loop(0, B // 16, body, 0)
            pltpu.sync_copy(ob_v, out_hbm.at[r])

    return k(idxT, t2)


def _dense_t(embT, gw1, gb1, gw2, gb2, tw1, tb1, tw2, tb2, tw3, tb3):
    BT = 2048
    grid = (B // BT,)
    c00 = (((0,), (0,)), ((), ()))

    def bcol(bias, shape):
        return lax.broadcast_in_dim(bias, shape, (0,))

    def dot00(a, b):
        return lax.dot_general(a, b, c00, preferred_element_type=jnp.float32)

    def body(embT_ref, gw1_ref, gb1_ref, gw2_ref, gb2_ref,
             tw1_ref, tb1_ref, tw2_ref, tb2_ref, tw3_ref, tb3_ref, out_ref):
        et = embT_ref[...]
        h = dot00(gw1_ref[:DOM, :], et[:DOM, :])
        h = h + dot00(gw1_ref[DOM:, :], et)
        h = jnp.maximum(h + bcol(gb1_ref[...], (GH, BT)), 0.0)
        g = dot00(gw2_ref[...], h)
        g = 2.0 * jax.nn.sigmoid(g + bcol(gb2_ref[...], (E, BT)))
        ep = jnp.tile(g, (F, 1)) * et
        outs = []
        for t in range(TN):
            h1 = dot00(tw1_ref[t], ep)
            h1 = jnp.maximum(h1 + bcol(tb1_ref[t], (512, BT)), 0.0)
            h2 = dot00(tw2_ref[t], h1)
            h2 = jnp.maximum(h2 + bcol(tb2_ref[t], (128, BT)), 0.0)
            lg = dot00(h2, tw3_ref[t])
            outs.append(lg + tb3_ref[t])
        out_ref[...] = jnp.concatenate(outs, axis=1)

    full = lambda *shape: pl.BlockSpec(shape, lambda i: (0,) * len(shape))
    return pl.pallas_call(
        body,
        grid=grid,
        in_specs=[
            pl.BlockSpec((GEN, BT), lambda i: (0, i)),
            full(*gw1.shape), full(*gb1.shape), full(*gw2.shape), full(*gb2.shape),
            full(*tw1.shape), full(*tb1.shape), full(*tw2.shape), full(*tb2.shape),
            full(*tw3.shape), full(*tb3.shape),
        ],
        out_specs=pl.BlockSpec((BT, TN), lambda i: (i, 0)),
        out_shape=jax.ShapeDtypeStruct((B, TN), jnp.float32),
        compiler_params=pltpu.CompilerParams(
            dimension_semantics=("arbitrary",),
        ),
    )(embT, gw1, gb1, gw2, gb2, tw1, tb1, tw2, tb2, tw3, tb3)


def kernel(inputs, tables, gate_W1, gate_b1, gate_W2, gate_b2,
           tower_W1, tower_b1, tower_W2, tower_b2, tower_W3, tower_b3):
    t2 = jnp.transpose(tables, (0, 2, 1)).reshape(GEN, V)
    idxT = jnp.transpose(inputs.astype(jnp.int32))
    embT = _sc_gather_t(idxT, t2)
    return _dense_t(embT, gate_W1, gate_b1, gate_W2, gate_b2,
                    tower_W1, tower_b1, tower_W2, tower_b2, tower_W3, tower_b3)
